# baseline (speedup 1.0000x reference)
import functools

import jax
import jax.numpy as jnp
from jax import lax
from jax.experimental import pallas as pl
from jax.experimental.pallas import tpu as pltpu
from jax.experimental.pallas import tpu_sc as plsc

N = 10000
E = 320000
D_IN = 128
H = 128
A_OUT = 8

NC = 2
NS = 16
NW = NC * NS
BB = 128
NB = 80
EPW = NB * BB
EPAD = NW * EPW
NP = 10240
NACC = N + 48
ROWCH = 16

_mesh = plsc.VectorSubcoreMesh(core_axis_name="c", subcore_axis_name="s")


@functools.partial(
    pl.kernel,
    out_type=jax.ShapeDtypeStruct((NC * NP,), jnp.float32),
    mesh=_mesh,
    scratch_types=[
        pltpu.VMEM_SHARED((NP,), jnp.float32),
        pltpu.VMEM((NB, BB), jnp.int32),
        pltpu.VMEM((128,), jnp.float32),
        pltpu.VMEM((16,), jnp.float32),
        pltpu.VMEM((640,), jnp.float32),
    ],
)
def _sc_degree(ei_hbm, out_hbm, acc, idx_v, ones_v, z16, tmp_v):
    c = lax.axis_index("c")
    s = lax.axis_index("s")
    w = c * NS + s

    @pl.loop(0, 128, step=16)
    def _(i):
        ones_v[pl.ds(i, 16)] = jnp.ones((16,), jnp.float32)

    z16[...] = jnp.zeros((16,), jnp.float32)

    @pl.loop(s * 16, NP, step=NS * 16)
    def _(off):
        pltpu.sync_copy(z16, acc.at[pl.ds(off, 16)])

    pltpu.sync_copy(ei_hbm.at[1, w], idx_v)
    plsc.subcore_barrier()

    @pl.loop(0, NB)
    def _(j):
        pltpu.sync_copy(ones_v, acc.at[idx_v.at[j]], add=True)

    plsc.subcore_barrier()

    pltpu.sync_copy(acc.at[pl.ds(s * 640, 640)], tmp_v)
    pltpu.sync_copy(tmp_v, out_hbm.at[pl.ds(c * NP + s * 640, 640)])


@functools.partial(
    pl.kernel,
    out_type=jax.ShapeDtypeStruct((NC, N, H), jnp.float32),
    mesh=_mesh,
    scratch_types=[
        pltpu.VMEM_SHARED((NACC, H), jnp.float32),
        pltpu.VMEM((NB // 2, BB), jnp.int32),
        pltpu.VMEM((NB // 2, BB), jnp.int32),
        pltpu.VMEM((BB, H), jnp.float32),
        pltpu.VMEM((BB, H), jnp.float32),
        pltpu.SemaphoreType.DMA,
        pltpu.SemaphoreType.DMA,
        pltpu.SemaphoreType.DMA,
        pltpu.SemaphoreType.DMA,
    ],
)
def _sc_aggregate(hs_hbm, ei_hbm, out_hbm,
                  acc, sidx, didx, bufa, bufb, sema, semb, ssema, ssemb):
    c = lax.axis_index("c")
    s = lax.axis_index("s")
    w = c * NS + s
    NBH = NB // 2
    ZR = 80

    @pl.loop(0, ZR)
    def _(r):
        @pl.loop(0, H, step=16)
        def _(cc):
            bufa[r, pl.ds(cc, 16)] = jnp.zeros((16,), jnp.float32)

    @pl.loop(s * ZR, N, step=NS * ZR)
    def _(r0):
        pltpu.async_copy(bufa.at[pl.ds(0, ZR)], acc.at[pl.ds(r0, ZR)], sema)

    @pl.loop(s * ZR, N, step=NS * ZR)
    def _(r0):
        pltpu.make_async_copy(bufa.at[pl.ds(0, ZR)], acc.at[pl.ds(r0, ZR)],
                              sema).wait()

    plsc.subcore_barrier()

    for half in range(2):
        pltpu.sync_copy(ei_hbm.at[0, w, pl.ds(half * NBH, NBH)], sidx)
        pltpu.sync_copy(ei_hbm.at[1, w, pl.ds(half * NBH, NBH)], didx)

        pltpu.async_copy(hs_hbm.at[sidx.at[0]], bufa, sema)
        pltpu.async_copy(hs_hbm.at[sidx.at[1]], bufb, semb)

        @pl.loop(0, NBH // 2)
        def _(t):
            j0 = 2 * t
            pltpu.make_async_copy(hs_hbm.at[sidx.at[j0]], bufa, sema).wait()
            d0 = pltpu.async_copy(bufa, acc.at[didx.at[j0]], ssema, add=True)
            pltpu.make_async_copy(hs_hbm.at[sidx.at[j0 + 1]], bufb, semb).wait()
            d0.wait()
            d1 = pltpu.async_copy(bufb, acc.at[didx.at[j0 + 1]], ssemb,
                                  add=True)

            @pl.when(j0 + 2 < NBH)
            def _():
                pltpu.async_copy(hs_hbm.at[sidx.at[j0 + 2]], bufa, sema)

            d1.wait()

            @pl.when(j0 + 3 < NBH)
            def _():
                pltpu.async_copy(hs_hbm.at[sidx.at[j0 + 3]], bufb, semb)

    plsc.subcore_barrier()

    NCH = (N + NS * ZR - 1) // (NS * ZR)
    for t in range(NCH):
        r0 = s * ZR + t * (NS * ZR)
        buf, sem = (bufa, sema) if t % 2 == 0 else (bufb, semb)

        @pl.when(r0 < N)
        def _(t=t, r0=r0, buf=buf, sem=sem):
            if t >= 2:
                rp = r0 - 2 * NS * ZR
                pltpu.make_async_copy(buf.at[pl.ds(0, ZR)],
                                      out_hbm.at[c, pl.ds(rp, ZR)], sem).wait()
            pltpu.sync_copy(acc.at[pl.ds(r0, ZR)], buf.at[pl.ds(0, ZR)])
            pltpu.async_copy(buf.at[pl.ds(0, ZR)],
                             out_hbm.at[c, pl.ds(r0, ZR)], sem)

    for t in (NCH - 2, NCH - 1):
        r0 = s * ZR + t * (NS * ZR)
        buf, sem = (bufa, sema) if t % 2 == 0 else (bufb, semb)

        @pl.when(r0 < N)
        def _(r0=r0, buf=buf, sem=sem):
            pltpu.make_async_copy(buf.at[pl.ds(0, ZR)],
                                  out_hbm.at[c, pl.ds(r0, ZR)], sem).wait()


_BLK = 2000


def _dis_expand_body(cnt_ref, out_ref):
    nr = NP // 128
    d2 = lax.rsqrt(cnt_ref[pl.ds(0, nr), :] + cnt_ref[pl.ds(nr, nr), :] + 1.0)
    eye = jnp.eye(128, dtype=jnp.float32)
    ones = jnp.ones((128, 128), jnp.float32)
    for i in range(nr):
        diag = eye * d2[i, :][None, :]
        out_ref[pl.ds(i * 128, 128), :] = jnp.dot(
            diag, ones, preferred_element_type=jnp.float32,
            precision=lax.Precision.HIGHEST)


def _dis_expand(cnt2):
    return pl.pallas_call(
        _dis_expand_body,
        in_specs=[pl.BlockSpec((2 * NP // 128, 128), lambda: (0, 0))],
        out_specs=pl.BlockSpec((NP, H), lambda: (0, 0)),
        out_shape=jax.ShapeDtypeStruct((NP, H), jnp.float32),
    )(cnt2)


def _tc_pre_body(x_ref, w_ref, dis_ref, hs_ref):
    hs_ref[...] = jnp.dot(x_ref[...], w_ref[...],
                          preferred_element_type=jnp.float32) * dis_ref[...]


def _tc_pre(x, W1, dis_b):
    return pl.pallas_call(
        _tc_pre_body,
        grid=(N // _BLK,),
        in_specs=[
            pl.BlockSpec((_BLK, D_IN), lambda i: (i, 0)),
            pl.BlockSpec((D_IN, H), lambda i: (0, 0)),
            pl.BlockSpec((_BLK, H), lambda i: (i, 0)),
        ],
        out_specs=pl.BlockSpec((_BLK, H), lambda i: (i, 0)),
        out_shape=jax.ShapeDtypeStruct((N, H), jnp.float32),
    )(x, W1, dis_b)


def _tc_mid_body(acc_ref, hs_ref, dis_ref, b_ref, w_ref, out_ref):
    dis = dis_ref[...]
    t = dis * (acc_ref[0] + acc_ref[1] + hs_ref[...]) + b_ref[...]
    t = jnp.maximum(t, 0.0)
    out_ref[...] = jnp.dot(t, w_ref[...],
                           preferred_element_type=jnp.float32) * dis


def _tc_mid(acc, hs, dis, b, W):
    return pl.pallas_call(
        _tc_mid_body,
        grid=(N // _BLK,),
        in_specs=[
            pl.BlockSpec((2, _BLK, H), lambda i: (0, i, 0)),
            pl.BlockSpec((_BLK, H), lambda i: (i, 0)),
            pl.BlockSpec((_BLK, H), lambda i: (i, 0)),
            pl.BlockSpec((1, H), lambda i: (0, 0)),
            pl.BlockSpec((H, H), lambda i: (0, 0)),
        ],
        out_specs=pl.BlockSpec((_BLK, H), lambda i: (i, 0)),
        out_shape=jax.ShapeDtypeStruct((N, H), jnp.float32),
    )(acc, hs, dis, b, W)


def _tc_head_body(acc_ref, hs_ref, dis_ref, b2_ref, wa1_ref, ba1_ref,
                  wa2_ref, ba2_ref, q_ref):
    h2 = dis_ref[...] * (acc_ref[0] + acc_ref[1] + hs_ref[...]) + b2_ref[...]
    h2 = jnp.maximum(h2, 0.0)
    t = jnp.maximum(
        jnp.dot(h2, wa1_ref[...], preferred_element_type=jnp.float32)
        + ba1_ref[...], 0.0)
    q_ref[...] = jnp.dot(t, wa2_ref[...],
                         preferred_element_type=jnp.float32) + ba2_ref[...]


def _tc_head(acc, hs, dis, b2, Wa1, ba1, Wa2, ba2):
    return pl.pallas_call(
        _tc_head_body,
        grid=(N // _BLK,),
        in_specs=[
            pl.BlockSpec((2, _BLK, H), lambda i: (0, i, 0)),
            pl.BlockSpec((_BLK, H), lambda i: (i, 0)),
            pl.BlockSpec((_BLK, H), lambda i: (i, 0)),
            pl.BlockSpec((1, H), lambda i: (0, 0)),
            pl.BlockSpec((H, H), lambda i: (0, 0)),
            pl.BlockSpec((1, H), lambda i: (0, 0)),
            pl.BlockSpec((H, A_OUT), lambda i: (0, 0)),
            pl.BlockSpec((1, A_OUT), lambda i: (0, 0)),
        ],
        out_specs=pl.BlockSpec((_BLK, A_OUT), lambda i: (i, 0)),
        out_shape=jax.ShapeDtypeStruct((N, A_OUT), jnp.float32),
    )(acc, hs, dis, b2, Wa1, ba1, Wa2, ba2)


def kernel(x, edge_index, W1, b1, W2, b2, Wa1, ba1, Wa2, ba2):
    pad = jnp.broadcast_to(
        jnp.array([[0], [N]], dtype=edge_index.dtype), (2, EPAD - E))
    ei = jnp.concatenate([edge_index, pad], axis=1).reshape(2, NW, NB, BB)

    cnt = _sc_degree(ei)
    dis_b = _dis_expand(cnt.reshape(2 * NP // 128, 128))
    hs1 = _tc_pre(x, W1, dis_b)
    acc1 = _sc_aggregate(hs1, ei)
    hs2 = _tc_mid(acc1, hs1, dis_b, b1.reshape(1, H), W2)
    acc2 = _sc_aggregate(hs2, ei)
    q = _tc_head(acc2, hs2, dis_b, b2.reshape(1, H), Wa1,
                 ba1.reshape(1, H), Wa2, ba2.reshape(1, A_OUT))
    return q


# speedup vs baseline: 1.1407x; 1.1407x over previous
import functools

import jax
import jax.numpy as jnp
from jax import lax
from jax.experimental import pallas as pl
from jax.experimental.pallas import tpu as pltpu
from jax.experimental.pallas import tpu_sc as plsc

N = 10000
E = 320000
D_IN = 128
H = 128
A_OUT = 8

NC = 2
NS = 16
NW = NC * NS
BB = 128
NB = 80
EPW = NB * BB
EPAD = NW * EPW
NP = 10240
NACC = N + 48
ROWCH = 16

_mesh = plsc.VectorSubcoreMesh(core_axis_name="c", subcore_axis_name="s")


@functools.partial(
    pl.kernel,
    out_type=jax.ShapeDtypeStruct((NC * NP,), jnp.float32),
    mesh=_mesh,
    scratch_types=[
        pltpu.VMEM_SHARED((NP,), jnp.float32),
        pltpu.VMEM((NB, BB), jnp.int32),
        pltpu.VMEM((128,), jnp.float32),
        pltpu.VMEM((16,), jnp.float32),
        pltpu.VMEM((640,), jnp.float32),
    ],
)
def _sc_degree(ei_hbm, out_hbm, acc, idx_v, ones_v, z16, tmp_v):
    c = lax.axis_index("c")
    s = lax.axis_index("s")
    w = c * NS + s

    @pl.loop(0, 128, step=16)
    def _(i):
        ones_v[pl.ds(i, 16)] = jnp.ones((16,), jnp.float32)

    z16[...] = jnp.zeros((16,), jnp.float32)

    @pl.loop(s * 16, NP, step=NS * 16)
    def _(off):
        pltpu.sync_copy(z16, acc.at[pl.ds(off, 16)])

    pltpu.sync_copy(ei_hbm.at[1, w], idx_v)
    plsc.subcore_barrier()

    @pl.loop(0, NB)
    def _(j):
        pltpu.sync_copy(ones_v, acc.at[idx_v.at[j]], add=True)

    plsc.subcore_barrier()

    pltpu.sync_copy(acc.at[pl.ds(s * 640, 640)], tmp_v)
    pltpu.sync_copy(tmp_v, out_hbm.at[pl.ds(c * NP + s * 640, 640)])


@functools.partial(
    pl.kernel,
    out_type=jax.ShapeDtypeStruct((NC, N, H), jnp.float32),
    mesh=_mesh,
    scratch_types=[
        pltpu.VMEM_SHARED((NACC, H), jnp.float32),
        pltpu.VMEM((NB // 2, BB), jnp.int32),
        pltpu.VMEM((NB // 2, BB), jnp.int32),
        pltpu.VMEM((BB, H), jnp.float32),
        pltpu.VMEM((BB, H), jnp.float32),
        pltpu.SemaphoreType.DMA,
        pltpu.SemaphoreType.DMA,
        pltpu.SemaphoreType.DMA,
        pltpu.SemaphoreType.DMA,
    ],
)
def _sc_aggregate(hs_hbm, ei_hbm, out_hbm,
                  acc, sidx, didx, bufa, bufb, sema, semb, ssema, ssemb):
    c = lax.axis_index("c")
    s = lax.axis_index("s")
    w = c * NS + s
    NBH = NB // 2
    ZR = 80

    @pl.loop(0, ZR)
    def _(r):
        @pl.loop(0, H, step=16)
        def _(cc):
            bufa[r, pl.ds(cc, 16)] = jnp.zeros((16,), jnp.float32)

    @pl.loop(s * ZR, N, step=NS * ZR)
    def _(r0):
        pltpu.async_copy(bufa.at[pl.ds(0, ZR)], acc.at[pl.ds(r0, ZR)], sema)

    @pl.loop(s * ZR, N, step=NS * ZR)
    def _(r0):
        pltpu.make_async_copy(bufa.at[pl.ds(0, ZR)], acc.at[pl.ds(r0, ZR)],
                              sema).wait()

    plsc.subcore_barrier()

    for half in range(2):
        pltpu.sync_copy(ei_hbm.at[0, w, pl.ds(half * NBH, NBH)], sidx)
        pltpu.sync_copy(ei_hbm.at[1, w, pl.ds(half * NBH, NBH)], didx)

        pltpu.async_copy(hs_hbm.at[sidx.at[0]], bufa, sema)
        pltpu.async_copy(hs_hbm.at[sidx.at[1]], bufb, semb)

        @pl.loop(0, NBH // 2)
        def _(t):
            j0 = 2 * t
            pltpu.make_async_copy(hs_hbm.at[sidx.at[j0]], bufa, sema).wait()
            d0 = pltpu.async_copy(bufa, acc.at[didx.at[j0]], ssema, add=True)
            pltpu.make_async_copy(hs_hbm.at[sidx.at[j0 + 1]], bufb, semb).wait()
            d0.wait()
            d1 = pltpu.async_copy(bufb, acc.at[didx.at[j0 + 1]], ssemb,
                                  add=True)

            @pl.when(j0 + 2 < NBH)
            def _():
                pltpu.async_copy(hs_hbm.at[sidx.at[j0 + 2]], bufa, sema)

            d1.wait()

            @pl.when(j0 + 3 < NBH)
            def _():
                pltpu.async_copy(hs_hbm.at[sidx.at[j0 + 3]], bufb, semb)

    plsc.subcore_barrier()

    NCH = (N + NS * ZR - 1) // (NS * ZR)
    for t in range(NCH):
        r0 = s * ZR + t * (NS * ZR)
        buf, sem = (bufa, sema) if t % 2 == 0 else (bufb, semb)

        @pl.when(r0 < N)
        def _(t=t, r0=r0, buf=buf, sem=sem):
            if t >= 2:
                rp = r0 - 2 * NS * ZR
                pltpu.make_async_copy(buf.at[pl.ds(0, ZR)],
                                      out_hbm.at[c, pl.ds(rp, ZR)], sem).wait()
            pltpu.sync_copy(acc.at[pl.ds(r0, ZR)], buf.at[pl.ds(0, ZR)])
            pltpu.async_copy(buf.at[pl.ds(0, ZR)],
                             out_hbm.at[c, pl.ds(r0, ZR)], sem)

    for t in (NCH - 2, NCH - 1):
        r0 = s * ZR + t * (NS * ZR)
        buf, sem = (bufa, sema) if t % 2 == 0 else (bufb, semb)

        @pl.when(r0 < N)
        def _(r0=r0, buf=buf, sem=sem):
            pltpu.make_async_copy(buf.at[pl.ds(0, ZR)],
                                  out_hbm.at[c, pl.ds(r0, ZR)], sem).wait()


_BLK = 2000


def _dis_expand_body(cnt_ref, out_ref):
    nr = NP // 128
    d2 = cnt_ref[pl.ds(0, nr), :] + cnt_ref[pl.ds(nr, nr), :] + 1.0
    eye = jnp.eye(128, dtype=jnp.float32)
    ones = jnp.ones((128, 128), jnp.float32)
    for i in range(nr):
        diag = eye * d2[i, :][None, :]
        deg_b = jnp.dot(diag, ones, preferred_element_type=jnp.float32,
                        precision=lax.Precision.HIGHEST)
        out_ref[pl.ds(i * 128, 128), :] = lax.rsqrt(deg_b)


def _dis_expand(cnt2):
    return pl.pallas_call(
        _dis_expand_body,
        in_specs=[pl.BlockSpec((2 * NP // 128, 128), lambda: (0, 0))],
        out_specs=pl.BlockSpec((NP, H), lambda: (0, 0)),
        out_shape=jax.ShapeDtypeStruct((NP, H), jnp.float32),
    )(cnt2)


def _tc_pre_body(x_ref, w_ref, dis_ref, hs_ref):
    hs_ref[...] = jnp.dot(x_ref[...], w_ref[...],
                          preferred_element_type=jnp.float32) * dis_ref[...]


def _tc_pre(x, W1, dis_b):
    return pl.pallas_call(
        _tc_pre_body,
        grid=(N // _BLK,),
        in_specs=[
            pl.BlockSpec((_BLK, D_IN), lambda i: (i, 0)),
            pl.BlockSpec((D_IN, H), lambda i: (0, 0)),
            pl.BlockSpec((_BLK, H), lambda i: (i, 0)),
        ],
        out_specs=pl.BlockSpec((_BLK, H), lambda i: (i, 0)),
        out_shape=jax.ShapeDtypeStruct((N, H), jnp.float32),
    )(x, W1, dis_b)


def _tc_mid_body(acc_ref, hs_ref, dis_ref, b_ref, w_ref, out_ref):
    dis = dis_ref[...]
    t = dis * (acc_ref[0] + acc_ref[1] + hs_ref[...]) + b_ref[...]
    t = jnp.maximum(t, 0.0)
    out_ref[...] = jnp.dot(t, w_ref[...],
                           preferred_element_type=jnp.float32) * dis


def _tc_mid(acc, hs, dis, b, W):
    return pl.pallas_call(
        _tc_mid_body,
        grid=(N // _BLK,),
        in_specs=[
            pl.BlockSpec((2, _BLK, H), lambda i: (0, i, 0)),
            pl.BlockSpec((_BLK, H), lambda i: (i, 0)),
            pl.BlockSpec((_BLK, H), lambda i: (i, 0)),
            pl.BlockSpec((1, H), lambda i: (0, 0)),
            pl.BlockSpec((H, H), lambda i: (0, 0)),
        ],
        out_specs=pl.BlockSpec((_BLK, H), lambda i: (i, 0)),
        out_shape=jax.ShapeDtypeStruct((N, H), jnp.float32),
    )(acc, hs, dis, b, W)


def _tc_head_body(acc_ref, hs_ref, dis_ref, b2_ref, wa1_ref, ba1_ref,
                  wa2_ref, ba2_ref, q_ref):
    h2 = dis_ref[...] * (acc_ref[0] + acc_ref[1] + hs_ref[...]) + b2_ref[...]
    h2 = jnp.maximum(h2, 0.0)
    t = jnp.maximum(
        jnp.dot(h2, wa1_ref[...], preferred_element_type=jnp.float32)
        + ba1_ref[...], 0.0)
    q_ref[...] = jnp.dot(t, wa2_ref[...],
                         preferred_element_type=jnp.float32) + ba2_ref[...]


def _tc_head(acc, hs, dis, b2, Wa1, ba1, Wa2, ba2):
    return pl.pallas_call(
        _tc_head_body,
        grid=(N // _BLK,),
        in_specs=[
            pl.BlockSpec((2, _BLK, H), lambda i: (0, i, 0)),
            pl.BlockSpec((_BLK, H), lambda i: (i, 0)),
            pl.BlockSpec((_BLK, H), lambda i: (i, 0)),
            pl.BlockSpec((1, H), lambda i: (0, 0)),
            pl.BlockSpec((H, H), lambda i: (0, 0)),
            pl.BlockSpec((1, H), lambda i: (0, 0)),
            pl.BlockSpec((H, A_OUT), lambda i: (0, 0)),
            pl.BlockSpec((1, A_OUT), lambda i: (0, 0)),
        ],
        out_specs=pl.BlockSpec((_BLK, A_OUT), lambda i: (i, 0)),
        out_shape=jax.ShapeDtypeStruct((N, A_OUT), jnp.float32),
    )(acc, hs, dis, b2, Wa1, ba1, Wa2, ba2)


def kernel(x, edge_index, W1, b1, W2, b2, Wa1, ba1, Wa2, ba2):
    padw = EPW - E // NW
    it = edge_index.dtype
    srcpad = jnp.zeros((1, NW, padw), it)
    dstpad = jnp.broadcast_to(N + (jnp.arange(padw, dtype=it) % 48),
                              (1, NW, padw))
    ei = jnp.concatenate(
        [edge_index.reshape(2, NW, E // NW),
         jnp.concatenate([srcpad, dstpad], axis=0)], axis=2,
    ).reshape(2, NW, NB, BB)

    cnt = _sc_degree(ei)
    dis_b = _dis_expand(cnt.reshape(2 * NP // 128, 128))
    hs1 = _tc_pre(x, W1, dis_b)
    acc1 = _sc_aggregate(hs1, ei)
    hs2 = _tc_mid(acc1, hs1, dis_b, b1.reshape(1, H), W2)
    acc2 = _sc_aggregate(hs2, ei)
    q = _tc_head(acc2, hs2, dis_b, b2.reshape(1, H), Wa1,
                 ba1.reshape(1, H), Wa2, ba2.reshape(1, A_OUT))
    return q


# speedup vs baseline: 3.3281x; 2.9176x over previous
import functools

import jax
import jax.numpy as jnp
from jax import lax
from jax.experimental import pallas as pl
from jax.experimental.pallas import tpu as pltpu
from jax.experimental.pallas import tpu_sc as plsc

N = 10000
E = 320000
D_IN = 128
H = 128
A_OUT = 8

NC = 2
NS = 16
NW = NC * NS
BB = 128
NB = 80
EPW = NB * BB
EPAD = NW * EPW
NP = 10240
NACC = N + 128
ROWCH = 16

_mesh = plsc.VectorSubcoreMesh(core_axis_name="c", subcore_axis_name="s")


@functools.partial(
    pl.kernel,
    out_type=jax.ShapeDtypeStruct((NC * NP,), jnp.float32),
    mesh=_mesh,
    scratch_types=[
        pltpu.VMEM_SHARED((NP,), jnp.float32),
        pltpu.VMEM((NB, BB), jnp.int32),
        pltpu.VMEM((128,), jnp.float32),
        pltpu.VMEM((16,), jnp.float32),
        pltpu.VMEM((640,), jnp.float32),
    ],
)
def _sc_degree(ei_hbm, out_hbm, acc, idx_v, ones_v, z16, tmp_v):
    c = lax.axis_index("c")
    s = lax.axis_index("s")
    w = c * NS + s

    @pl.loop(0, 128, step=16)
    def _(i):
        ones_v[pl.ds(i, 16)] = jnp.ones((16,), jnp.float32)

    z16[...] = jnp.zeros((16,), jnp.float32)

    @pl.loop(s * 16, NP, step=NS * 16)
    def _(off):
        pltpu.sync_copy(z16, acc.at[pl.ds(off, 16)])

    pltpu.sync_copy(ei_hbm.at[1, w], idx_v)
    plsc.subcore_barrier()

    @pl.loop(0, NB)
    def _(j):
        pltpu.sync_copy(ones_v, acc.at[idx_v.at[j]], add=True)

    plsc.subcore_barrier()

    pltpu.sync_copy(acc.at[pl.ds(s * 640, 640)], tmp_v)
    pltpu.sync_copy(tmp_v, out_hbm.at[pl.ds(c * NP + s * 640, 640)])


@functools.partial(
    pl.kernel,
    out_type=jax.ShapeDtypeStruct((NC, N, H), jnp.float32),
    mesh=_mesh,
    scratch_types=[
        pltpu.VMEM_SHARED((NACC, H), jnp.float32),
        pltpu.VMEM((NB // 2, BB), jnp.int32),
        pltpu.VMEM((NB // 2, BB), jnp.int32),
        pltpu.VMEM((BB, H), jnp.float32),
        pltpu.VMEM((BB, H), jnp.float32),
        pltpu.SemaphoreType.DMA,
        pltpu.SemaphoreType.DMA,
        pltpu.SemaphoreType.DMA,
        pltpu.SemaphoreType.DMA,
    ],
)
def _sc_aggregate(hs_hbm, ei_hbm, out_hbm,
                  acc, sidx, didx, bufa, bufb, sema, semb, ssema, ssemb):
    c = lax.axis_index("c")
    s = lax.axis_index("s")
    w = c * NS + s
    NBH = NB // 2
    ZR = 80

    @pl.loop(0, ZR)
    def _(r):
        @pl.loop(0, H, step=16)
        def _(cc):
            bufa[r, pl.ds(cc, 16)] = jnp.zeros((16,), jnp.float32)

    @pl.loop(s * ZR, N, step=NS * ZR)
    def _(r0):
        pltpu.async_copy(bufa.at[pl.ds(0, ZR)], acc.at[pl.ds(r0, ZR)], sema)

    @pl.loop(s * ZR, N, step=NS * ZR)
    def _(r0):
        pltpu.make_async_copy(bufa.at[pl.ds(0, ZR)], acc.at[pl.ds(r0, ZR)],
                              sema).wait()

    plsc.subcore_barrier()

    for half in range(2):
        pltpu.sync_copy(ei_hbm.at[0, w, pl.ds(half * NBH, NBH)], sidx)
        pltpu.sync_copy(ei_hbm.at[1, w, pl.ds(half * NBH, NBH)], didx)

        pltpu.async_copy(hs_hbm.at[sidx.at[0]], bufa, sema)
        pltpu.async_copy(hs_hbm.at[sidx.at[1]], bufb, semb)

        @pl.loop(0, NBH // 2)
        def _(t):
            j0 = 2 * t
            pltpu.make_async_copy(hs_hbm.at[sidx.at[j0]], bufa, sema).wait()
            d0 = pltpu.async_copy(bufa, acc.at[didx.at[j0]], ssema, add=True)
            pltpu.make_async_copy(hs_hbm.at[sidx.at[j0 + 1]], bufb, semb).wait()
            d0.wait()
            d1 = pltpu.async_copy(bufb, acc.at[didx.at[j0 + 1]], ssemb,
                                  add=True)

            @pl.when(j0 + 2 < NBH)
            def _():
                pltpu.async_copy(hs_hbm.at[sidx.at[j0 + 2]], bufa, sema)

            d1.wait()

            @pl.when(j0 + 3 < NBH)
            def _():
                pltpu.async_copy(hs_hbm.at[sidx.at[j0 + 3]], bufb, semb)

    plsc.subcore_barrier()

    NCH = (N + NS * ZR - 1) // (NS * ZR)
    for t in range(NCH):
        r0 = s * ZR + t * (NS * ZR)
        buf, sem = (bufa, sema) if t % 2 == 0 else (bufb, semb)

        @pl.when(r0 < N)
        def _(t=t, r0=r0, buf=buf, sem=sem):
            if t >= 2:
                rp = r0 - 2 * NS * ZR
                pltpu.make_async_copy(buf.at[pl.ds(0, ZR)],
                                      out_hbm.at[c, pl.ds(rp, ZR)], sem).wait()
            pltpu.sync_copy(acc.at[pl.ds(r0, ZR)], buf.at[pl.ds(0, ZR)])
            pltpu.async_copy(buf.at[pl.ds(0, ZR)],
                             out_hbm.at[c, pl.ds(r0, ZR)], sem)

    for t in (NCH - 2, NCH - 1):
        r0 = s * ZR + t * (NS * ZR)
        buf, sem = (bufa, sema) if t % 2 == 0 else (bufb, semb)

        @pl.when(r0 < N)
        def _(r0=r0, buf=buf, sem=sem):
            pltpu.make_async_copy(buf.at[pl.ds(0, ZR)],
                                  out_hbm.at[c, pl.ds(r0, ZR)], sem).wait()


_BLK = 2000


def _dis_expand_body(cnt_ref, out_ref):
    nr = NP // 128
    d2 = cnt_ref[pl.ds(0, nr), :] + cnt_ref[pl.ds(nr, nr), :] + 1.0
    eye = jnp.eye(128, dtype=jnp.float32)
    ones = jnp.ones((128, 128), jnp.float32)
    for i in range(nr):
        diag = eye * d2[i, :][None, :]
        deg_b = jnp.dot(diag, ones, preferred_element_type=jnp.float32,
                        precision=lax.Precision.HIGHEST)
        out_ref[pl.ds(i * 128, 128), :] = lax.rsqrt(deg_b)


def _dis_expand(cnt2):
    return pl.pallas_call(
        _dis_expand_body,
        in_specs=[pl.BlockSpec((2 * NP // 128, 128), lambda: (0, 0))],
        out_specs=pl.BlockSpec((NP, H), lambda: (0, 0)),
        out_shape=jax.ShapeDtypeStruct((NP, H), jnp.float32),
    )(cnt2)


def _tc_pre_body(x_ref, w_ref, dis_ref, hs_ref):
    hs_ref[...] = jnp.dot(x_ref[...], w_ref[...],
                          preferred_element_type=jnp.float32) * dis_ref[...]


def _tc_pre(x, W1, dis_b):
    return pl.pallas_call(
        _tc_pre_body,
        grid=(N // _BLK,),
        in_specs=[
            pl.BlockSpec((_BLK, D_IN), lambda i: (i, 0)),
            pl.BlockSpec((D_IN, H), lambda i: (0, 0)),
            pl.BlockSpec((_BLK, H), lambda i: (i, 0)),
        ],
        out_specs=pl.BlockSpec((_BLK, H), lambda i: (i, 0)),
        out_shape=jax.ShapeDtypeStruct((N, H), jnp.float32),
    )(x, W1, dis_b)


def _tc_mid_body(acc_ref, hs_ref, dis_ref, b_ref, w_ref, out_ref):
    dis = dis_ref[...]
    t = dis * (acc_ref[0] + acc_ref[1] + hs_ref[...]) + b_ref[...]
    t = jnp.maximum(t, 0.0)
    out_ref[...] = jnp.dot(t, w_ref[...],
                           preferred_element_type=jnp.float32) * dis


def _tc_mid(acc, hs, dis, b, W):
    return pl.pallas_call(
        _tc_mid_body,
        grid=(N // _BLK,),
        in_specs=[
            pl.BlockSpec((2, _BLK, H), lambda i: (0, i, 0)),
            pl.BlockSpec((_BLK, H), lambda i: (i, 0)),
            pl.BlockSpec((_BLK, H), lambda i: (i, 0)),
            pl.BlockSpec((1, H), lambda i: (0, 0)),
            pl.BlockSpec((H, H), lambda i: (0, 0)),
        ],
        out_specs=pl.BlockSpec((_BLK, H), lambda i: (i, 0)),
        out_shape=jax.ShapeDtypeStruct((N, H), jnp.float32),
    )(acc, hs, dis, b, W)


def _tc_head_body(acc_ref, hs_ref, dis_ref, b2_ref, wa1_ref, ba1_ref,
                  wa2_ref, ba2_ref, q_ref):
    h2 = dis_ref[...] * (acc_ref[0] + acc_ref[1] + hs_ref[...]) + b2_ref[...]
    h2 = jnp.maximum(h2, 0.0)
    t = jnp.maximum(
        jnp.dot(h2, wa1_ref[...], preferred_element_type=jnp.float32)
        + ba1_ref[...], 0.0)
    q_ref[...] = jnp.dot(t, wa2_ref[...],
                         preferred_element_type=jnp.float32) + ba2_ref[...]


def _tc_head(acc, hs, dis, b2, Wa1, ba1, Wa2, ba2):
    return pl.pallas_call(
        _tc_head_body,
        grid=(N // _BLK,),
        in_specs=[
            pl.BlockSpec((2, _BLK, H), lambda i: (0, i, 0)),
            pl.BlockSpec((_BLK, H), lambda i: (i, 0)),
            pl.BlockSpec((_BLK, H), lambda i: (i, 0)),
            pl.BlockSpec((1, H), lambda i: (0, 0)),
            pl.BlockSpec((H, H), lambda i: (0, 0)),
            pl.BlockSpec((1, H), lambda i: (0, 0)),
            pl.BlockSpec((H, A_OUT), lambda i: (0, 0)),
            pl.BlockSpec((1, A_OUT), lambda i: (0, 0)),
        ],
        out_specs=pl.BlockSpec((_BLK, A_OUT), lambda i: (i, 0)),
        out_shape=jax.ShapeDtypeStruct((N, A_OUT), jnp.float32),
    )(acc, hs, dis, b2, Wa1, ba1, Wa2, ba2)


def kernel(x, edge_index, W1, b1, W2, b2, Wa1, ba1, Wa2, ba2):
    padw = EPW - E // NW
    it = edge_index.dtype
    seq = jnp.arange(padw, dtype=it) % 128
    srcpad = jnp.broadcast_to(seq, (1, NW, padw))
    dstpad = jnp.broadcast_to(N + seq, (1, NW, padw))
    ei = jnp.concatenate(
        [edge_index.reshape(2, NW, E // NW),
         jnp.concatenate([srcpad, dstpad], axis=0)], axis=2,
    ).reshape(2, NW, NB, BB)

    cnt = _sc_degree(ei)
    dis_b = _dis_expand(cnt.reshape(2 * NP // 128, 128))
    hs1 = _tc_pre(x, W1, dis_b)
    acc1 = _sc_aggregate(hs1, ei)
    hs2 = _tc_mid(acc1, hs1, dis_b, b1.reshape(1, H), W2)
    acc2 = _sc_aggregate(hs2, ei)
    q = _tc_head(acc2, hs2, dis_b, b2.reshape(1, H), Wa1,
                 ba1.reshape(1, H), Wa2, ba2.reshape(1, A_OUT))
    return q
